# baseline (speedup 1.0000x reference)
import functools

import jax
import jax.numpy as jnp
from jax import lax
from jax.experimental import pallas as pl
from jax.experimental.pallas import tpu as pltpu
from jax.experimental.pallas import tpu_sc as plsc

DP = 1024
SP = 56


def _make_gather(V, D, BATCH, SEQ):
    info = plsc.get_sparse_core_info()
    NC, NS = info.num_cores, info.num_subcores
    NW = NC * NS
    assert BATCH % NW == 0
    n_chunks = BATCH // NW
    assert n_chunks % 2 == 0 and n_chunks >= 4

    mesh = plsc.VectorSubcoreMesh(core_axis_name="c", subcore_axis_name="s")

    @functools.partial(
        pl.kernel,
        out_type=jax.ShapeDtypeStruct((BATCH, SP, 8, DP // 8), jnp.float32),
        mesh=mesh,
        scratch_types=[
            pltpu.VMEM((n_chunks, 1, SP), jnp.int32),
            pltpu.VMEM((SP, 8, DP // 8), jnp.float32),
            pltpu.VMEM((SP, 8, DP // 8), jnp.float32),
            pltpu.SemaphoreType.DMA,
            pltpu.SemaphoreType.DMA,
            pltpu.SemaphoreType.DMA,
            pltpu.SemaphoreType.DMA,
        ],
        compiler_params=pltpu.CompilerParams(use_tc_tiling_on_sc=True),
    )
    def gather_k(table_hbm, idx_hbm, out_hbm, idx_v, buf0, buf1,
                 gsem0, gsem1, osem0, osem1):
        wid = lax.axis_index("s") * NC + lax.axis_index("c")
        pltpu.sync_copy(idx_hbm.at[wid], idx_v)
        out_base = wid * n_chunks

        bufs = (buf0, buf1)
        gsems = (gsem0, gsem1)
        osems = (osem0, osem1)

        def g_start(j, b):
            pltpu.async_copy(table_hbm.at[idx_v.at[j, 0]], bufs[b], gsems[b])

        def g_wait(j, b):
            pltpu.make_async_copy(
                table_hbm.at[idx_v.at[j, 0]], bufs[b], gsems[b]).wait()

        def o_start(j, b):
            pltpu.async_copy(bufs[b], out_hbm.at[out_base + j], osems[b])

        def o_wait(j, b):
            pltpu.make_async_copy(bufs[b], out_hbm.at[out_base + j],
                                  osems[b]).wait()

        g_start(0, 0)
        g_wait(0, 0)
        o_start(0, 0)
        g_start(1, 1)

        def pair(i, carry):
            j0 = 2 * i
            j1 = j0 + 1
            g_wait(j0 - 1, 1)
            o_wait(j0 - 2, 0)
            o_start(j0 - 1, 1)
            g_start(j0, 0)
            g_wait(j0, 0)
            o_wait(j1 - 2, 1)
            o_start(j0, 0)
            g_start(j1, 1)
            return carry

        lax.fori_loop(1, n_chunks // 2, pair, 0)

        last = n_chunks - 1
        g_wait(last, 1)
        o_wait(last - 1, 0)
        o_start(last, 1)
        o_wait(last, 1)

    return gather_k


def kernel(x, table):
    BATCH, SEQ = x.shape
    V, D = table.shape
    NW = 32
    table_p = jnp.pad(table, ((0, 0), (0, DP - D))).reshape(V, 8, DP // 8)
    x_p = jnp.pad(x, ((0, 0), (0, SP - SEQ)))
    idx4d = x_p.reshape(NW, BATCH // NW, 1, SP).astype(jnp.int32)
    out_p = _make_gather(V, D, BATCH, SEQ)(table_p, idx4d)
    return out_p.reshape(BATCH, SP, DP)[:, :SEQ, :D]


# speedup vs baseline: 1.4196x; 1.4196x over previous
import functools

import jax
import jax.numpy as jnp
from jax import lax
from jax.experimental import pallas as pl
from jax.experimental.pallas import tpu as pltpu
from jax.experimental.pallas import tpu_sc as plsc


def _make_gather(V, D, BATCH, SEQ):
    info = plsc.get_sparse_core_info()
    NC, NS = info.num_cores, info.num_subcores
    NW = NC * NS
    assert BATCH % NW == 0
    n_chunks = BATCH // NW
    assert n_chunks % 2 == 0 and n_chunks >= 4

    mesh = plsc.VectorSubcoreMesh(core_axis_name="c", subcore_axis_name="s")

    @functools.partial(
        pl.kernel,
        out_type=jax.ShapeDtypeStruct((BATCH, SEQ, D), jnp.float32),
        mesh=mesh,
        scratch_types=[
            pltpu.VMEM((n_chunks, SEQ), jnp.int32),
            pltpu.VMEM((SEQ, D), jnp.float32),
            pltpu.VMEM((SEQ, D), jnp.float32),
            pltpu.SemaphoreType.DMA,
            pltpu.SemaphoreType.DMA,
            pltpu.SemaphoreType.DMA,
            pltpu.SemaphoreType.DMA,
        ],
        compiler_params=pltpu.CompilerParams(use_tc_tiling_on_sc=False),
    )
    def gather_k(table_hbm, idx_hbm, out_hbm, idx_v, buf0, buf1,
                 gsem0, gsem1, osem0, osem1):
        wid = lax.axis_index("s") * NC + lax.axis_index("c")
        pltpu.sync_copy(idx_hbm.at[wid], idx_v)
        out_base = wid * n_chunks

        bufs = (buf0, buf1)
        gsems = (gsem0, gsem1)
        osems = (osem0, osem1)

        def g_start(j, b):
            pltpu.async_copy(table_hbm.at[idx_v.at[j]], bufs[b], gsems[b])

        def g_wait(j, b):
            pltpu.make_async_copy(
                table_hbm.at[idx_v.at[j]], bufs[b], gsems[b]).wait()

        def o_start(j, b):
            pltpu.async_copy(bufs[b], out_hbm.at[out_base + j], osems[b])

        def o_wait(j, b):
            pltpu.make_async_copy(
                bufs[b], out_hbm.at[out_base + j], osems[b]).wait()

        g_start(0, 0)
        g_wait(0, 0)
        o_start(0, 0)
        g_start(1, 1)

        def pair(i, carry):
            j0 = 2 * i
            j1 = j0 + 1
            g_wait(j0 - 1, 1)
            o_wait(j0 - 2, 0)
            o_start(j0 - 1, 1)
            g_start(j0, 0)
            g_wait(j0, 0)
            o_wait(j1 - 2, 1)
            o_start(j0, 0)
            g_start(j1, 1)
            return carry

        lax.fori_loop(1, n_chunks // 2, pair, 0)

        last = n_chunks - 1
        g_wait(last, 1)
        o_wait(last - 1, 0)
        o_start(last, 1)
        o_wait(last, 1)

    return gather_k


def kernel(x, table):
    BATCH, SEQ = x.shape
    V, D = table.shape
    NW = 32
    idx3d = x.reshape(NW, BATCH // NW, SEQ).astype(jnp.int32)
    return _make_gather(V, D, BATCH, SEQ)(table, idx3d)
